# Initial kernel scaffold; baseline (speedup 1.0000x reference)
import jax
import jax.numpy as jnp
from jax.experimental import pallas as pl


def kernel(x, router_w, router_b, w1, b1, w2, b2):
    raise NotImplementedError("write your pallas kernel here")


# speedup vs baseline: 1.0720x; 1.0720x over previous
import jax
import jax.numpy as jnp
from jax.experimental import pallas as pl
from jax.experimental.pallas import tpu as pltpu

_HIDDEN = 768
_NUM_EXPERTS = 8
_TOP_K = 2
_D_FF = _HIDDEN * 2


def _moe_dense_body(x_ref, rw_ref, rb_ref, w1_ref, b1_ref, w2_ref, b2_ref, o_ref):
    e = pl.program_id(0)
    x = x_ref[...]
    logits = jnp.dot(x, rw_ref[...], preferred_element_type=jnp.float32) + rb_ref[...][None, :]
    idx = jax.lax.broadcasted_iota(jnp.int32, logits.shape, 1)
    m1 = jnp.max(logits, axis=1, keepdims=True)
    i1 = jnp.min(jnp.where(logits == m1, idx, _NUM_EXPERTS), axis=1, keepdims=True)
    l2 = jnp.where(idx == i1, -jnp.inf, logits)
    m2 = jnp.max(l2, axis=1, keepdims=True)
    i2 = jnp.min(jnp.where(l2 == m2, idx, _NUM_EXPERTS), axis=1, keepdims=True)
    r = jnp.exp(m2 - m1)
    w_top1 = 1.0 / (1.0 + r)
    w_top2 = r / (1.0 + r)
    w_e = jnp.where(i1 == e, w_top1, 0.0) + jnp.where(i2 == e, w_top2, 0.0)

    h = jnp.maximum(
        jnp.dot(x, w1_ref[0], preferred_element_type=jnp.float32) + b1_ref[0], 0.0)
    eo = jnp.dot(h, w2_ref[0], preferred_element_type=jnp.float32) + b2_ref[0]
    contrib = eo * w_e

    @pl.when(e == 0)
    def _init():
        o_ref[...] = contrib

    @pl.when(e > 0)
    def _acc():
        o_ref[...] += contrib


def kernel(x, router_w, router_b, w1, b1, w2, b2):
    B, S, H = x.shape
    T = B * S
    xf = x.reshape(T, H)
    out = pl.pallas_call(
        _moe_dense_body,
        grid=(_NUM_EXPERTS,),
        in_specs=[
            pl.BlockSpec((T, H), lambda e: (0, 0)),
            pl.BlockSpec((H, _NUM_EXPERTS), lambda e: (0, 0)),
            pl.BlockSpec((_NUM_EXPERTS,), lambda e: (0,)),
            pl.BlockSpec((1, H, _D_FF), lambda e: (e, 0, 0)),
            pl.BlockSpec((1, 1, _D_FF), lambda e: (e, 0, 0)),
            pl.BlockSpec((1, _D_FF, H), lambda e: (e, 0, 0)),
            pl.BlockSpec((1, 1, H), lambda e: (e, 0, 0)),
        ],
        out_specs=pl.BlockSpec((T, H), lambda e: (0, 0)),
        out_shape=jax.ShapeDtypeStruct((T, H), jnp.float32),
        compiler_params=pltpu.CompilerParams(
            dimension_semantics=("arbitrary",),
        ),
    )(xf, router_w, router_b, w1,
      b1.reshape(_NUM_EXPERTS, 1, _D_FF), w2, b2.reshape(_NUM_EXPERTS, 1, H))
    return out.reshape(B, S, H)
